# baseline (speedup 1.0000x reference)
import functools

import jax
import jax.numpy as jnp
from jax import lax
from jax.experimental import pallas as pl
from jax.experimental.pallas import tpu as pltpu
from jax.experimental.pallas import tpu_sc as plsc

N = 10000
E = 160000
H = 128
L = 5
PRED = 16
NF = 4
NGRAPH = 16
EPS = 1e-8

NWORK = 32
CHUNK = 80
NCHUNK = E // CHUNK
ITERS = -(-NCHUNK // NWORK)
NSLICE = 4
ES = E // NSLICE
NCHUNK_S = ES // CHUNK
ITERS_S = 16
ITERS2_S = 32
NB = 1000
EB = 1000
NBLK = N // NB

f32 = jnp.float32
i32 = jnp.int32


def _silu(x):
    return x * (1.0 / (1.0 + jnp.exp(-x)))


def _pipeline(nwork, wid, niters, nchunk, start, finish):
    def guarded(j, fn, ph):
        k = j * jnp.int32(nwork) + wid

        @pl.when((j < niters) & (k < nchunk))
        def _():
            fn(k * jnp.int32(CHUNK), ph)

    guarded(jnp.int32(0), start, 0)

    def body(jj, carry):
        j0 = jj * jnp.int32(2)
        guarded(j0 + 1, start, 1)
        guarded(j0, finish, 0)
        guarded(j0 + 2, start, 0)
        guarded(j0 + 1, finish, 1)
        return carry

    lax.fori_loop(jnp.int32(0), jnp.int32(niters // 2), body, jnp.int32(0))


def _sc_hgather_body(nchunk, niters, ta_hbm, tb_hbm, row_hbm, col_hbm,
                     ga_hbm, gb_hbm,
                     idxr_v, idxc_v, bufa_v, bufb_v, sa0, sa1, sb0, sb1):
    wid = (lax.axis_index("s").astype(i32) * jnp.int32(2)
           + lax.axis_index("c").astype(i32))
    sems_a = (sa0, sa1)
    sems_b = (sb0, sb1)

    def start(base, ph):
        pltpu.sync_copy(row_hbm.at[pl.ds(base, CHUNK)], idxr_v.at[jnp.int32(ph)])
        pltpu.sync_copy(col_hbm.at[pl.ds(base, CHUNK)], idxc_v.at[jnp.int32(ph)])
        pltpu.async_copy(ta_hbm.at[idxr_v.at[jnp.int32(ph)]], bufa_v.at[jnp.int32(ph)], sems_a[ph])
        pltpu.async_copy(tb_hbm.at[idxc_v.at[jnp.int32(ph)]], bufb_v.at[jnp.int32(ph)], sems_b[ph])

    def finish(base, ph):
        pltpu.make_async_copy(ta_hbm.at[idxr_v.at[jnp.int32(ph)]], bufa_v.at[jnp.int32(ph)],
                              sems_a[ph]).wait()
        pltpu.make_async_copy(tb_hbm.at[idxc_v.at[jnp.int32(ph)]], bufb_v.at[jnp.int32(ph)],
                              sems_b[ph]).wait()
        pltpu.sync_copy(bufa_v.at[jnp.int32(ph)], ga_hbm.at[pl.ds(base, CHUNK)])
        pltpu.sync_copy(bufb_v.at[jnp.int32(ph)], gb_hbm.at[pl.ds(base, CHUNK)])

    _pipeline(NWORK, wid, niters, nchunk, start, finish)


def _sc_cgather_body(nchunk, niters, tc_hbm, row_hbm, col_hbm, cd_hbm,
                     idxr_v, idxc_v, bufr_v, bufc_v, sr0, sr1, sc0, sc1):
    wid = (lax.axis_index("s").astype(i32) * jnp.int32(2)
           + lax.axis_index("c").astype(i32))
    sems_r = (sr0, sr1)
    sems_c = (sc0, sc1)

    def start(base, ph):
        pltpu.sync_copy(row_hbm.at[pl.ds(base, CHUNK)], idxr_v.at[jnp.int32(ph)])
        pltpu.sync_copy(col_hbm.at[pl.ds(base, CHUNK)], idxc_v.at[jnp.int32(ph)])
        pltpu.async_copy(tc_hbm.at[idxr_v.at[jnp.int32(ph)]], bufr_v.at[jnp.int32(ph)], sems_r[ph])
        pltpu.async_copy(tc_hbm.at[idxc_v.at[jnp.int32(ph)]], bufc_v.at[jnp.int32(ph)], sems_c[ph])

    def finish(base, ph):
        pltpu.make_async_copy(tc_hbm.at[idxr_v.at[jnp.int32(ph)]], bufr_v.at[jnp.int32(ph)],
                              sems_r[ph]).wait()
        pltpu.make_async_copy(tc_hbm.at[idxc_v.at[jnp.int32(ph)]], bufc_v.at[jnp.int32(ph)],
                              sems_c[ph]).wait()

        def sub_row(r, carry):
            a = bufr_v[jnp.int32(ph), r, pl.ds(0, 16)]
            b = bufc_v[jnp.int32(ph), r, pl.ds(0, 16)]
            bufr_v[jnp.int32(ph), r, pl.ds(0, 16)] = a - b
            return carry

        lax.fori_loop(jnp.int32(0), jnp.int32(CHUNK), sub_row, jnp.int32(0))
        pltpu.sync_copy(bufr_v.at[jnp.int32(ph)], cd_hbm.at[pl.ds(base, CHUNK)])

    _pipeline(NWORK, wid, niters, nchunk, start, finish)


def _writeout(sid, src_sh, dst_hbm):
    @pl.when(sid < 15)
    def _():
        off = sid * jnp.int32(624)
        pltpu.sync_copy(src_sh.at[pl.ds(off, 624)],
                        dst_hbm.at[pl.ds(off, 624)])

    @pl.when(sid == 15)
    def _():
        pltpu.sync_copy(src_sh.at[pl.ds(9360, 640)],
                        dst_hbm.at[pl.ds(jnp.int32(9360), 640)])


def _sc_scatter2_body(nchunk, niters, feat_hbm, trans_hbm, row_hbm, zh_hbm,
                      agg_hbm, s_hbm, idx_v, dat_v, acc_sh, s0, s1):
    cid = lax.axis_index("c").astype(i32)
    sid = lax.axis_index("s").astype(i32)
    sems = (s0, s1)

    @pl.when(sid == 0)
    def _():
        pltpu.sync_copy(zh_hbm, acc_sh)

    plsc.subcore_barrier()

    def run(src_hbm):
        def start(base, ph):
            pltpu.sync_copy(row_hbm.at[pl.ds(base, CHUNK)], idx_v.at[jnp.int32(ph)])
            pltpu.async_copy(src_hbm.at[pl.ds(base, CHUNK)], dat_v.at[jnp.int32(ph)],
                             sems[ph])

        def finish(base, ph):
            pltpu.make_async_copy(src_hbm.at[pl.ds(base, CHUNK)],
                                  dat_v.at[jnp.int32(ph)], sems[ph]).wait()
            pltpu.sync_copy(dat_v.at[jnp.int32(ph)], acc_sh.at[idx_v.at[jnp.int32(ph)]], add=True)

        _pipeline(16, sid, niters, nchunk, start, finish)

    @pl.when(cid == 0)
    def _():
        run(feat_hbm)

    @pl.when(cid == 1)
    def _():
        run(trans_hbm)

    plsc.subcore_barrier()

    @pl.when(cid == 0)
    def _():
        _writeout(sid, acc_sh, agg_hbm)

    @pl.when(cid == 1)
    def _():
        _writeout(sid, acc_sh, s_hbm)


def _sc_count_body(row_hbm, ones_hbm, zh_hbm, cnt_hbm, idx_v, ones_v, cnt_sh):
    cid = lax.axis_index("c").astype(i32)
    sid = lax.axis_index("s").astype(i32)
    wid = sid * jnp.int32(2) + cid

    @pl.when(sid == 0)
    def _():
        pltpu.sync_copy(zh_hbm, cnt_sh)

    pltpu.sync_copy(ones_hbm, ones_v)
    plsc.subcore_barrier()

    def body(j, carry):
        k = j * jnp.int32(NWORK) + wid

        @pl.when(k < NCHUNK)
        def _():
            base = k * jnp.int32(CHUNK)
            pltpu.sync_copy(row_hbm.at[pl.ds(base, CHUNK)], idx_v)
            pltpu.sync_copy(ones_v, cnt_sh.at[idx_v], add=True)

        return carry

    lax.fori_loop(jnp.int32(0), jnp.int32(ITERS), body, jnp.int32(0))
    plsc.subcore_barrier()

    @pl.when(cid == 0)
    def _():
        _writeout(sid, cnt_sh, cnt_hbm.at[pl.ds(0, N)])

    @pl.when(cid == 1)
    def _():
        _writeout(sid, cnt_sh, cnt_hbm.at[pl.ds(N, N)])


@functools.cache
def _sc_kernels():
    mesh = plsc.VectorSubcoreMesh(core_axis_name="c", subcore_axis_name="s",
                                  num_cores=2, num_subcores=16)
    hgather = pl.kernel(
        functools.partial(_sc_hgather_body, NCHUNK_S, ITERS_S),
        out_type=(
            jax.ShapeDtypeStruct((ES, H), f32),
            jax.ShapeDtypeStruct((ES, H), f32),
        ),
        mesh=mesh,
        scratch_types=[
            pltpu.VMEM((2, CHUNK), i32),
            pltpu.VMEM((2, CHUNK), i32),
            pltpu.VMEM((2, CHUNK, H), f32),
            pltpu.VMEM((2, CHUNK, H), f32),
            pltpu.SemaphoreType.DMA,
            pltpu.SemaphoreType.DMA,
            pltpu.SemaphoreType.DMA,
            pltpu.SemaphoreType.DMA,
        ],
    )
    cgather = pl.kernel(
        functools.partial(_sc_cgather_body, NCHUNK_S, ITERS_S),
        out_type=jax.ShapeDtypeStruct((ES, H), f32),
        mesh=mesh,
        scratch_types=[
            pltpu.VMEM((2, CHUNK), i32),
            pltpu.VMEM((2, CHUNK), i32),
            pltpu.VMEM((2, CHUNK, H), f32),
            pltpu.VMEM((2, CHUNK, H), f32),
            pltpu.SemaphoreType.DMA,
            pltpu.SemaphoreType.DMA,
            pltpu.SemaphoreType.DMA,
            pltpu.SemaphoreType.DMA,
        ],
    )
    scatter2 = pl.kernel(
        functools.partial(_sc_scatter2_body, NCHUNK_S, ITERS2_S),
        out_type=(
            jax.ShapeDtypeStruct((N, H), f32),
            jax.ShapeDtypeStruct((N, H), f32),
        ),
        mesh=mesh,
        scratch_types=[
            pltpu.VMEM((2, CHUNK), i32),
            pltpu.VMEM((2, CHUNK, H), f32),
            pltpu.VMEM_SHARED((N, H), f32),
            pltpu.SemaphoreType.DMA,
            pltpu.SemaphoreType.DMA,
        ],
    )
    count = pl.kernel(
        _sc_count_body,
        out_type=jax.ShapeDtypeStruct((2 * N, H), f32),
        mesh=mesh,
        scratch_types=[
            pltpu.VMEM((CHUNK,), i32),
            pltpu.VMEM((CHUNK, H), f32),
            pltpu.VMEM_SHARED((N, H), f32),
        ],
    )
    return hgather, cgather, scatter2, count


def _sc_hgather(ta, tb, row, col):
    return _sc_kernels()[0](ta, tb, row, col)


def _sc_cgather(coordp, row, col):
    return _sc_kernels()[1](coordp, row, col)


def _sc_scatter2(feat, trans, row, zh):
    return _sc_kernels()[2](feat, trans, row, zh)


def _sc_count(row, ones_chunk, zh):
    return _sc_kernels()[3](row, ones_chunk, zh)


def _full(shape):
    return pl.BlockSpec(shape, lambda i: tuple(jnp.int32(0) for _ in shape))


def _rowblk(cols):
    return pl.BlockSpec((NB, cols), lambda i: (i, jnp.int32(0)))


def _rowblk2(cols):
    return pl.BlockSpec((NB, cols),
                        lambda i: (i + jnp.int32(NBLK), jnp.int32(0)))


def _node_init_body(xf_ref, wemb_ref, bemb_ref, sc_ref, bi_ref,
                    a_ref, b_ref, h_ref, ta_ref, tb_ref):
    h0 = jnp.dot(xf_ref[...], wemb_ref[...], preferred_element_type=f32)
    h = sc_ref[...] * (h0 + bemb_ref[...]) + bi_ref[...]
    h_ref[...] = h
    ta_ref[...] = jnp.dot(h, a_ref[...], preferred_element_type=f32)
    tb_ref[...] = jnp.dot(h, b_ref[...], preferred_element_type=f32)


def _node_init(xf, wemb, bemb, scale, bias, a_w, b_w):
    return pl.pallas_call(
        _node_init_body,
        grid=(NBLK,),
        in_specs=[
            _rowblk(PRED * NF),
            _full((PRED * NF, H)),
            _full((1, H)),
            _full((1, H)),
            _full((1, H)),
            _full((H, H)),
            _full((H, H)),
        ],
        out_specs=[_rowblk(H), _rowblk(H), _rowblk(H)],
        out_shape=[
            jax.ShapeDtypeStruct((N, H), f32),
            jax.ShapeDtypeStruct((N, H), f32),
            jax.ShapeDtypeStruct((N, H), f32),
        ],
    )(xf, wemb, bemb, scale, bias, a_w, b_w)


def _edge_body(ga_ref, gb_ref, cd_ref, ea_ref, uvec_ref, eew_ref,
               eeb_ref, w1he_ref, b1_ref, w2_ref, b2_ref, cw1_ref, cb1_ref,
               cw2_ref, feat_ref, trans_ref):
    hpart = ga_ref[...] + gb_ref[...]
    cd = cd_ref[...]
    radial = jnp.sum(cd * cd, axis=1, keepdims=True)
    norm = jnp.sqrt(radial) + EPS
    cdn = cd / norm
    w1he = w1he_ref[...]
    vvec = jnp.dot(eew_ref[...], w1he, preferred_element_type=f32)
    cvec = jnp.dot(eeb_ref[...], w1he, preferred_element_type=f32) + b1_ref[...]
    pre1 = hpart + radial * uvec_ref[...] + ea_ref[...] * vvec + cvec
    f1 = _silu(pre1)
    f2 = _silu(jnp.dot(f1, w2_ref[...], preferred_element_type=f32)
               + b2_ref[...])
    hc = _silu(jnp.dot(f2, cw1_ref[...], preferred_element_type=f32)
               + cb1_ref[...])
    cm = jnp.sum(hc * cw2_ref[...], axis=1, keepdims=True)
    feat_ref[...] = f2
    trans_ref[...] = cdn * cm


def _edge_mlp(ga, gb, cd, ea2, uvec, eew, eeb, w1he, b1, w2, b2,
              cw1, cb1, cw2r):
    ne = ga.shape[0]
    eb = pl.BlockSpec((EB, H), lambda i: (i, jnp.int32(0)))
    eb1 = pl.BlockSpec((EB, 1), lambda i: (i, jnp.int32(0)))
    return pl.pallas_call(
        _edge_body,
        grid=(ne // EB,),
        in_specs=[
            eb, eb, eb, eb1,
            _full((1, H)), _full((1, H)), _full((1, H)),
            _full((H, H)), _full((1, H)),
            _full((H, H)), _full((1, H)),
            _full((H, H)), _full((1, H)), _full((1, H)),
        ],
        out_specs=[eb, eb],
        out_shape=[
            jax.ShapeDtypeStruct((ne, H), f32),
            jax.ShapeDtypeStruct((ne, H), f32),
        ],
    )(ga, gb, cd, ea2, uvec, eew, eeb, w1he, b1, w2, b2, cw1, cb1, cw2r)


def _node_mid_body(h_ref, agg0_ref, agg1_ref, agg2_ref, agg3_ref,
                   s0_ref, s1_ref, s2_ref, s3_ref, cnt0_ref,
                   cnt1_ref, coord_ref,
                   nw1h_ref, nw1a_ref, nb1_ref, nw2_ref, nb2_ref,
                   sc_ref, bi_ref, a_ref, b_ref,
                   hn_ref, coordn_ref, ta_ref, tb_ref):
    h = h_ref[...]
    agg = (agg0_ref[...] + agg1_ref[...]) + (agg2_ref[...] + agg3_ref[...])
    ssum = (s0_ref[...] + s1_ref[...]) + (s2_ref[...] + s3_ref[...])
    cnt = cnt0_ref[...] + cnt1_ref[...]
    coordn_ref[...] = coord_ref[...] + ssum / jnp.maximum(cnt, 1.0)
    t = _silu(jnp.dot(h, nw1h_ref[...], preferred_element_type=f32)
              + jnp.dot(agg, nw1a_ref[...], preferred_element_type=f32)
              + nb1_ref[...])
    out = jnp.dot(t, nw2_ref[...], preferred_element_type=f32) + nb2_ref[...]
    hn = sc_ref[...] * (h + out) + bi_ref[...]
    hn_ref[...] = hn
    ta_ref[...] = jnp.dot(hn, a_ref[...], preferred_element_type=f32)
    tb_ref[...] = jnp.dot(hn, b_ref[...], preferred_element_type=f32)


def _node_mid(h, aggs, ss, cntp, coord, nw1h, nw1a, nb1, nw2, nb2,
              scale, bias, a_w, b_w):
    return pl.pallas_call(
        _node_mid_body,
        grid=(NBLK,),
        in_specs=[
            _rowblk(H),
            _rowblk(H), _rowblk(H), _rowblk(H), _rowblk(H),
            _rowblk(H), _rowblk(H), _rowblk(H), _rowblk(H),
            _rowblk(H), _rowblk2(H),
            _rowblk(H),
            _full((H, H)), _full((H, H)), _full((1, H)),
            _full((H, H)), _full((1, H)),
            _full((1, H)), _full((1, H)),
            _full((H, H)), _full((H, H)),
        ],
        out_specs=[_rowblk(H), _rowblk(H), _rowblk(H), _rowblk(H)],
        out_shape=[
            jax.ShapeDtypeStruct((N, H), f32),
            jax.ShapeDtypeStruct((N, H), f32),
            jax.ShapeDtypeStruct((N, H), f32),
            jax.ShapeDtypeStruct((N, H), f32),
        ],
    )(h, *aggs, *ss, cntp, cntp, coord, nw1h, nw1a, nb1, nw2, nb2,
      scale, bias, a_w, b_w)


def _node_last_body(h_ref, agg0_ref, agg1_ref, agg2_ref, agg3_ref,
                    s0_ref, s1_ref, s2_ref, s3_ref, cnt0_ref,
                    cnt1_ref, coord_ref,
                    nw1h_ref, nw1a_ref, nb1_ref, nw2_ref, nb2_ref, batch_ref,
                    hn_ref, coordn_ref, gsum_ref, gcnt_ref):
    i = pl.program_id(0)
    h = h_ref[...]
    agg = (agg0_ref[...] + agg1_ref[...]) + (agg2_ref[...] + agg3_ref[...])
    ssum = (s0_ref[...] + s1_ref[...]) + (s2_ref[...] + s3_ref[...])
    cnt = cnt0_ref[...] + cnt1_ref[...]
    coordn_ref[...] = coord_ref[...] + ssum / jnp.maximum(cnt, 1.0)
    t = _silu(jnp.dot(h, nw1h_ref[...], preferred_element_type=f32)
              + jnp.dot(agg, nw1a_ref[...], preferred_element_type=f32)
              + nb1_ref[...])
    hend = (h + jnp.dot(t, nw2_ref[...], preferred_element_type=f32)
            + nb2_ref[...])
    hn_ref[...] = hend
    iota = lax.broadcasted_iota(i32, (NB, NGRAPH), 1)
    mask = (batch_ref[...] == iota).astype(f32)
    gsum_part = lax.dot_general(mask, hend, (((0,), (0,)), ((), ())),
                                preferred_element_type=f32)
    gcnt_part = jnp.sum(mask, axis=0, keepdims=True)

    @pl.when(i == 0)
    def _():
        gsum_ref[...] = jnp.zeros_like(gsum_ref)
        gcnt_ref[...] = jnp.zeros_like(gcnt_ref)

    gsum_ref[...] += gsum_part
    gcnt_ref[...] += gcnt_part


def _node_last(h, aggs, ss, cntp, coord, nw1h, nw1a, nb1, nw2,
               nb2, batch2d):
    return pl.pallas_call(
        _node_last_body,
        grid=(NBLK,),
        in_specs=[
            _rowblk(H),
            _rowblk(H), _rowblk(H), _rowblk(H), _rowblk(H),
            _rowblk(H), _rowblk(H), _rowblk(H), _rowblk(H),
            _rowblk(H), _rowblk2(H),
            _rowblk(H),
            _full((H, H)), _full((H, H)), _full((1, H)),
            _full((H, H)), _full((1, H)),
            _rowblk(1),
        ],
        out_specs=[
            _rowblk(H),
            _rowblk(H),
            pl.BlockSpec((NGRAPH, H), lambda i: (jnp.int32(0), jnp.int32(0))),
            pl.BlockSpec((1, NGRAPH), lambda i: (jnp.int32(0), jnp.int32(0))),
        ],
        out_shape=[
            jax.ShapeDtypeStruct((N, H), f32),
            jax.ShapeDtypeStruct((N, H), f32),
            jax.ShapeDtypeStruct((NGRAPH, H), f32),
            jax.ShapeDtypeStruct((1, NGRAPH), f32),
        ],
    )(h, *aggs, *ss, cntp, cntp, coord, nw1h, nw1a, nb1, nw2, nb2,
      batch2d)


def _pred_body(h_ref, gsum_ref, gcnt_ref, batch_ref, pw1g_ref, pw1h_ref,
               pb1_ref, pw2_ref, pb2_ref, pw3_ref, pb3_ref, out_ref):
    g = gsum_ref[...] / jnp.maximum(gcnt_ref[...].reshape(NGRAPH, 1), 1.0)
    iota = lax.broadcasted_iota(i32, (NB, NGRAPH), 1)
    mask = (batch_ref[...] == iota).astype(f32)
    gnode = jnp.dot(mask, g, preferred_element_type=f32)
    t = jnp.maximum(jnp.dot(gnode, pw1g_ref[...], preferred_element_type=f32)
                    + jnp.dot(h_ref[...], pw1h_ref[...],
                              preferred_element_type=f32)
                    + pb1_ref[...], 0.0)
    t = jnp.maximum(jnp.dot(t, pw2_ref[...], preferred_element_type=f32)
                    + pb2_ref[...], 0.0)
    out_ref[...] = (jnp.dot(t, pw3_ref[...], preferred_element_type=f32)
                    + pb3_ref[...])


def _pred_mlp(hend, gsum, gcnt, batch2d, pw1g, pw1h, pb1, pw2, pb2, pw3, pb3):
    return pl.pallas_call(
        _pred_body,
        grid=(NBLK,),
        in_specs=[
            _rowblk(H),
            _full((NGRAPH, H)),
            _full((1, NGRAPH)),
            _rowblk(1),
            _full((H, H)), _full((H, H)), _full((1, H)),
            _full((H, H)), _full((1, H)),
            _full((H, PRED * NF)), _full((1, PRED * NF)),
        ],
        out_specs=_rowblk(PRED * NF),
        out_shape=jax.ShapeDtypeStruct((N, PRED * NF), f32),
    )(hend, gsum, gcnt, batch2d, pw1g, pw1h, pb1, pw2, pb2, pw3, pb3)


def kernel(x, edge_attr, x_coord, film_cond, node_emb_w, node_emb_b,
           edge_emb_w, edge_emb_b, l_edge_w1, l_edge_b1, l_edge_w2, l_edge_b2,
           l_node_w1, l_node_b1, l_node_w2, l_node_b2, l_coord_w1, l_coord_b1,
           l_coord_w2, pred_w1, pred_b1, pred_w2, pred_b2, pred_w3, pred_b3,
           edge_index, batch):
    row = edge_index[0].astype(i32)
    col = edge_index[1].astype(i32)
    batch2d = batch.astype(i32).reshape(N, 1)
    xf = x.reshape(N, PRED * NF).astype(f32)
    ea2 = edge_attr.astype(f32).reshape(E, 1)
    coordp = jnp.zeros((N, H), f32).at[:, :3].set(x_coord.astype(f32))

    embed = film_cond.astype(f32).reshape(L, 2, H)
    scales = embed[:, 0, :]
    biases = embed[:, 1, :]

    zh = jnp.zeros((N, H), f32)
    ones_chunk = jnp.ones((CHUNK, H), f32)

    cntp = _sc_count(row, ones_chunk, zh)

    def r1(v):
        return v.astype(f32).reshape(1, -1)

    h, ta, tb = _node_init(
        xf, node_emb_w.astype(f32), r1(node_emb_b), scales[0:1], biases[0:1],
        l_edge_w1[0, :H].astype(f32), l_edge_w1[0, H:2 * H].astype(f32))
    coord = coordp

    rows_s = [row[i * ES:(i + 1) * ES] for i in range(NSLICE)]
    cols_s = [col[i * ES:(i + 1) * ES] for i in range(NSLICE)]
    eas_s = [ea2[i * ES:(i + 1) * ES] for i in range(NSLICE)]

    for l in range(L):
        w1 = l_edge_w1[l].astype(f32)
        ew = (w1[2 * H:2 * H + 1, :], edge_emb_w.astype(f32), r1(edge_emb_b),
              w1[2 * H + 1:, :], r1(l_edge_b1[l]),
              l_edge_w2[l].astype(f32), r1(l_edge_b2[l]),
              l_coord_w1[l].astype(f32), r1(l_coord_b1[l]),
              l_coord_w2[l].astype(f32).reshape(1, H))
        gs = []
        for i in range(NSLICE):
            ga_i, gb_i = _sc_hgather(ta, tb, rows_s[i], cols_s[i])
            cd_i = _sc_cgather(coord, rows_s[i], cols_s[i])
            gs.append((ga_i, gb_i, cd_i))
        fts = [_edge_mlp(ga_i, gb_i, cd_i, eas_s[i], *ew)
               for i, (ga_i, gb_i, cd_i) in enumerate(gs)]
        parts = [_sc_scatter2(feat_i, trans_i, rows_s[i], zh)
                 for i, (feat_i, trans_i) in enumerate(fts)]
        aggs = [p[0] for p in parts]
        ss = [p[1] for p in parts]
        nw1 = l_node_w1[l].astype(f32)
        if l < L - 1:
            h, coord, ta, tb = _node_mid(
                h, aggs, ss, cntp, coord,
                nw1[:H], nw1[H:], r1(l_node_b1[l]),
                l_node_w2[l].astype(f32), r1(l_node_b2[l]),
                scales[l + 1:l + 2], biases[l + 1:l + 2],
                l_edge_w1[l + 1, :H].astype(f32),
                l_edge_w1[l + 1, H:2 * H].astype(f32))
        else:
            hend, coord, gsum, gcnt = _node_last(
                h, aggs, ss, cntp, coord,
                nw1[:H], nw1[H:], r1(l_node_b1[l]),
                l_node_w2[l].astype(f32), r1(l_node_b2[l]), batch2d)

    p = _pred_mlp(hend, gsum, gcnt, batch2d,
                  pred_w1[:H].astype(f32), pred_w1[H:].astype(f32),
                  r1(pred_b1), pred_w2.astype(f32), r1(pred_b2),
                  pred_w3.astype(f32), r1(pred_b3))
    return p.reshape(N, PRED, NF), coord[:, :3]


# speedup vs baseline: 1.1993x; 1.1993x over previous
import functools

import jax
import jax.numpy as jnp
from jax import lax
from jax.experimental import pallas as pl
from jax.experimental.pallas import tpu as pltpu
from jax.experimental.pallas import tpu_sc as plsc

N = 10000
E = 160000
H = 128
L = 5
PRED = 16
NF = 4
NGRAPH = 16
EPS = 1e-8

NWORK = 32
CHUNK = 128
NCHUNK = E // CHUNK
ITERS = -(-NCHUNK // NWORK)
EH = E // 2
NCHUNK_H = EH // CHUNK
ITERS_H = 20
ITERS2_H = 40
NB = 1000
EB = 2000
NBLK = N // NB

f32 = jnp.float32
i32 = jnp.int32


def _silu(x):
    return x * (1.0 / (1.0 + jnp.exp(-x)))


def _pipeline(nwork, wid, niters, nchunk, start, finish):
    def guarded(j, fn, ph):
        k = j * jnp.int32(nwork) + wid

        @pl.when((j < niters) & (k < nchunk))
        def _():
            fn(k * jnp.int32(CHUNK), ph)

    guarded(jnp.int32(0), start, 0)

    def body(jj, carry):
        j0 = jj * jnp.int32(2)
        guarded(j0 + 1, start, 1)
        guarded(j0, finish, 0)
        guarded(j0 + 2, start, 0)
        guarded(j0 + 1, finish, 1)
        return carry

    lax.fori_loop(jnp.int32(0), jnp.int32(niters // 2), body, jnp.int32(0))


def _sc_hgather_body(nchunk, niters, ta_hbm, tb_hbm, row_hbm, col_hbm,
                     ga_hbm, gb_hbm,
                     idxr_v, idxc_v, bufa_v, bufb_v, sa0, sa1, sb0, sb1):
    wid = (lax.axis_index("s").astype(i32) * jnp.int32(2)
           + lax.axis_index("c").astype(i32))
    sems_a = (sa0, sa1)
    sems_b = (sb0, sb1)

    def start(base, ph):
        pltpu.sync_copy(row_hbm.at[pl.ds(base, CHUNK)], idxr_v.at[jnp.int32(ph)])
        pltpu.sync_copy(col_hbm.at[pl.ds(base, CHUNK)], idxc_v.at[jnp.int32(ph)])
        pltpu.async_copy(ta_hbm.at[idxr_v.at[jnp.int32(ph)]], bufa_v.at[jnp.int32(ph)], sems_a[ph])
        pltpu.async_copy(tb_hbm.at[idxc_v.at[jnp.int32(ph)]], bufb_v.at[jnp.int32(ph)], sems_b[ph])

    def finish(base, ph):
        pltpu.make_async_copy(ta_hbm.at[idxr_v.at[jnp.int32(ph)]], bufa_v.at[jnp.int32(ph)],
                              sems_a[ph]).wait()
        pltpu.make_async_copy(tb_hbm.at[idxc_v.at[jnp.int32(ph)]], bufb_v.at[jnp.int32(ph)],
                              sems_b[ph]).wait()
        pltpu.sync_copy(bufa_v.at[jnp.int32(ph)], ga_hbm.at[pl.ds(base, CHUNK)])
        pltpu.sync_copy(bufb_v.at[jnp.int32(ph)], gb_hbm.at[pl.ds(base, CHUNK)])

    _pipeline(NWORK, wid, niters, nchunk, start, finish)


def _sc_cgather_body(nchunk, niters, tc_hbm, row_hbm, col_hbm, cd_hbm,
                     idxr_v, idxc_v, bufr_v, bufc_v, sr0, sr1, sc0, sc1):
    wid = (lax.axis_index("s").astype(i32) * jnp.int32(2)
           + lax.axis_index("c").astype(i32))
    sems_r = (sr0, sr1)
    sems_c = (sc0, sc1)

    def start(base, ph):
        pltpu.sync_copy(row_hbm.at[pl.ds(base, CHUNK)], idxr_v.at[jnp.int32(ph)])
        pltpu.sync_copy(col_hbm.at[pl.ds(base, CHUNK)], idxc_v.at[jnp.int32(ph)])
        pltpu.async_copy(tc_hbm.at[idxr_v.at[jnp.int32(ph)]], bufr_v.at[jnp.int32(ph)], sems_r[ph])
        pltpu.async_copy(tc_hbm.at[idxc_v.at[jnp.int32(ph)]], bufc_v.at[jnp.int32(ph)], sems_c[ph])

    def finish(base, ph):
        pltpu.make_async_copy(tc_hbm.at[idxr_v.at[jnp.int32(ph)]], bufr_v.at[jnp.int32(ph)],
                              sems_r[ph]).wait()
        pltpu.make_async_copy(tc_hbm.at[idxc_v.at[jnp.int32(ph)]], bufc_v.at[jnp.int32(ph)],
                              sems_c[ph]).wait()

        def sub_row(r, carry):
            a = bufr_v[jnp.int32(ph), r, pl.ds(0, 16)]
            b = bufc_v[jnp.int32(ph), r, pl.ds(0, 16)]
            bufr_v[jnp.int32(ph), r, pl.ds(0, 16)] = a - b
            return carry

        lax.fori_loop(jnp.int32(0), jnp.int32(CHUNK), sub_row, jnp.int32(0))
        pltpu.sync_copy(bufr_v.at[jnp.int32(ph)], cd_hbm.at[pl.ds(base, CHUNK)])

    _pipeline(NWORK, wid, niters, nchunk, start, finish)


def _writeout(sid, src_sh, dst_hbm):
    @pl.when(sid < 15)
    def _():
        off = sid * jnp.int32(624)
        pltpu.sync_copy(src_sh.at[pl.ds(off, 624)],
                        dst_hbm.at[pl.ds(off, 624)])

    @pl.when(sid == 15)
    def _():
        pltpu.sync_copy(src_sh.at[pl.ds(9360, 640)],
                        dst_hbm.at[pl.ds(jnp.int32(9360), 640)])


def _sc_scatter2_body(nchunk, niters, feat_hbm, trans_hbm, row_hbm, zh_hbm,
                      agg_hbm, s_hbm, idx_v, dat_v, acc_sh, s0, s1):
    cid = lax.axis_index("c").astype(i32)
    sid = lax.axis_index("s").astype(i32)
    sems = (s0, s1)

    @pl.when(sid == 0)
    def _():
        pltpu.sync_copy(zh_hbm, acc_sh)

    plsc.subcore_barrier()

    def run(src_hbm):
        def start(base, ph):
            pltpu.sync_copy(row_hbm.at[pl.ds(base, CHUNK)], idx_v.at[jnp.int32(ph)])
            pltpu.async_copy(src_hbm.at[pl.ds(base, CHUNK)], dat_v.at[jnp.int32(ph)],
                             sems[ph])

        def finish(base, ph):
            pltpu.make_async_copy(src_hbm.at[pl.ds(base, CHUNK)],
                                  dat_v.at[jnp.int32(ph)], sems[ph]).wait()
            pltpu.sync_copy(dat_v.at[jnp.int32(ph)], acc_sh.at[idx_v.at[jnp.int32(ph)]], add=True)

        _pipeline(16, sid, niters, nchunk, start, finish)

    @pl.when(cid == 0)
    def _():
        run(feat_hbm)

    @pl.when(cid == 1)
    def _():
        run(trans_hbm)

    plsc.subcore_barrier()

    @pl.when(cid == 0)
    def _():
        _writeout(sid, acc_sh, agg_hbm)

    @pl.when(cid == 1)
    def _():
        _writeout(sid, acc_sh, s_hbm)


def _sc_count_body(row_hbm, ones_hbm, zh_hbm, cnt_hbm, idx_v, ones_v, cnt_sh):
    cid = lax.axis_index("c").astype(i32)
    sid = lax.axis_index("s").astype(i32)
    wid = sid * jnp.int32(2) + cid

    @pl.when(sid == 0)
    def _():
        pltpu.sync_copy(zh_hbm, cnt_sh)

    pltpu.sync_copy(ones_hbm, ones_v)
    plsc.subcore_barrier()

    def body(j, carry):
        k = j * jnp.int32(NWORK) + wid

        @pl.when(k < NCHUNK)
        def _():
            base = k * jnp.int32(CHUNK)
            pltpu.sync_copy(row_hbm.at[pl.ds(base, CHUNK)], idx_v)
            pltpu.sync_copy(ones_v, cnt_sh.at[idx_v], add=True)

        return carry

    lax.fori_loop(jnp.int32(0), jnp.int32(ITERS), body, jnp.int32(0))
    plsc.subcore_barrier()

    @pl.when(cid == 0)
    def _():
        _writeout(sid, cnt_sh, cnt_hbm.at[pl.ds(0, N)])

    @pl.when(cid == 1)
    def _():
        _writeout(sid, cnt_sh, cnt_hbm.at[pl.ds(N, N)])


@functools.cache
def _sc_kernels():
    mesh = plsc.VectorSubcoreMesh(core_axis_name="c", subcore_axis_name="s",
                                  num_cores=2, num_subcores=16)
    hgather = pl.kernel(
        functools.partial(_sc_hgather_body, NCHUNK_H, ITERS_H),
        out_type=(
            jax.ShapeDtypeStruct((EH, H), f32),
            jax.ShapeDtypeStruct((EH, H), f32),
        ),
        mesh=mesh,
        scratch_types=[
            pltpu.VMEM((2, CHUNK), i32),
            pltpu.VMEM((2, CHUNK), i32),
            pltpu.VMEM((2, CHUNK, H), f32),
            pltpu.VMEM((2, CHUNK, H), f32),
            pltpu.SemaphoreType.DMA,
            pltpu.SemaphoreType.DMA,
            pltpu.SemaphoreType.DMA,
            pltpu.SemaphoreType.DMA,
        ],
    )
    cgather = pl.kernel(
        functools.partial(_sc_cgather_body, NCHUNK_H, ITERS_H),
        out_type=jax.ShapeDtypeStruct((EH, H), f32),
        mesh=mesh,
        scratch_types=[
            pltpu.VMEM((2, CHUNK), i32),
            pltpu.VMEM((2, CHUNK), i32),
            pltpu.VMEM((2, CHUNK, H), f32),
            pltpu.VMEM((2, CHUNK, H), f32),
            pltpu.SemaphoreType.DMA,
            pltpu.SemaphoreType.DMA,
            pltpu.SemaphoreType.DMA,
            pltpu.SemaphoreType.DMA,
        ],
    )
    scatter2 = pl.kernel(
        functools.partial(_sc_scatter2_body, NCHUNK_H, ITERS2_H),
        out_type=(
            jax.ShapeDtypeStruct((N, H), f32),
            jax.ShapeDtypeStruct((N, H), f32),
        ),
        mesh=mesh,
        scratch_types=[
            pltpu.VMEM((2, CHUNK), i32),
            pltpu.VMEM((2, CHUNK, H), f32),
            pltpu.VMEM_SHARED((N, H), f32),
            pltpu.SemaphoreType.DMA,
            pltpu.SemaphoreType.DMA,
        ],
    )
    count = pl.kernel(
        _sc_count_body,
        out_type=jax.ShapeDtypeStruct((2 * N, H), f32),
        mesh=mesh,
        scratch_types=[
            pltpu.VMEM((CHUNK,), i32),
            pltpu.VMEM((CHUNK, H), f32),
            pltpu.VMEM_SHARED((N, H), f32),
        ],
    )
    return hgather, cgather, scatter2, count


def _sc_hgather(ta, tb, row, col):
    return _sc_kernels()[0](ta, tb, row, col)


def _sc_cgather(coordp, row, col):
    return _sc_kernels()[1](coordp, row, col)


def _sc_scatter2(feat, trans, row, zh):
    return _sc_kernels()[2](feat, trans, row, zh)


def _sc_count(row, ones_chunk, zh):
    return _sc_kernels()[3](row, ones_chunk, zh)


def _full(shape):
    return pl.BlockSpec(shape, lambda i: tuple(jnp.int32(0) for _ in shape))


def _rowblk(cols):
    return pl.BlockSpec((NB, cols), lambda i: (i, jnp.int32(0)))


def _rowblk2(cols):
    return pl.BlockSpec((NB, cols),
                        lambda i: (i + jnp.int32(NBLK), jnp.int32(0)))


def _node_init_body(xf_ref, wemb_ref, bemb_ref, sc_ref, bi_ref,
                    a_ref, b_ref, h_ref, ta_ref, tb_ref):
    h0 = jnp.dot(xf_ref[...], wemb_ref[...], preferred_element_type=f32)
    h = sc_ref[...] * (h0 + bemb_ref[...]) + bi_ref[...]
    h_ref[...] = h
    ta_ref[...] = jnp.dot(h, a_ref[...], preferred_element_type=f32)
    tb_ref[...] = jnp.dot(h, b_ref[...], preferred_element_type=f32)


def _node_init(xf, wemb, bemb, scale, bias, a_w, b_w):
    return pl.pallas_call(
        _node_init_body,
        grid=(NBLK,),
        in_specs=[
            _rowblk(PRED * NF),
            _full((PRED * NF, H)),
            _full((1, H)),
            _full((1, H)),
            _full((1, H)),
            _full((H, H)),
            _full((H, H)),
        ],
        out_specs=[_rowblk(H), _rowblk(H), _rowblk(H)],
        out_shape=[
            jax.ShapeDtypeStruct((N, H), f32),
            jax.ShapeDtypeStruct((N, H), f32),
            jax.ShapeDtypeStruct((N, H), f32),
        ],
    )(xf, wemb, bemb, scale, bias, a_w, b_w)


def _edge_body(ga_ref, gb_ref, cd_ref, ea_ref, uvec_ref, eew_ref,
               eeb_ref, w1he_ref, b1_ref, w2_ref, b2_ref, cw1_ref, cb1_ref,
               cw2_ref, feat_ref, trans_ref):
    hpart = ga_ref[...] + gb_ref[...]
    cd = cd_ref[...]
    radial = jnp.sum(cd * cd, axis=1, keepdims=True)
    norm = jnp.sqrt(radial) + EPS
    cdn = cd / norm
    w1he = w1he_ref[...]
    vvec = jnp.dot(eew_ref[...], w1he, preferred_element_type=f32)
    cvec = jnp.dot(eeb_ref[...], w1he, preferred_element_type=f32) + b1_ref[...]
    pre1 = hpart + radial * uvec_ref[...] + ea_ref[...] * vvec + cvec
    f1 = _silu(pre1)
    f2 = _silu(jnp.dot(f1, w2_ref[...], preferred_element_type=f32)
               + b2_ref[...])
    hc = _silu(jnp.dot(f2, cw1_ref[...], preferred_element_type=f32)
               + cb1_ref[...])
    cm = jnp.sum(hc * cw2_ref[...], axis=1, keepdims=True)
    feat_ref[...] = f2
    trans_ref[...] = cdn * cm


def _edge_mlp(ga, gb, cd, ea2, uvec, eew, eeb, w1he, b1, w2, b2,
              cw1, cb1, cw2r):
    ne = ga.shape[0]
    eb = pl.BlockSpec((EB, H), lambda i: (i, jnp.int32(0)))
    eb1 = pl.BlockSpec((EB, 1), lambda i: (i, jnp.int32(0)))
    return pl.pallas_call(
        _edge_body,
        grid=(ne // EB,),
        in_specs=[
            eb, eb, eb, eb1,
            _full((1, H)), _full((1, H)), _full((1, H)),
            _full((H, H)), _full((1, H)),
            _full((H, H)), _full((1, H)),
            _full((H, H)), _full((1, H)), _full((1, H)),
        ],
        out_specs=[eb, eb],
        out_shape=[
            jax.ShapeDtypeStruct((ne, H), f32),
            jax.ShapeDtypeStruct((ne, H), f32),
        ],
    )(ga, gb, cd, ea2, uvec, eew, eeb, w1he, b1, w2, b2, cw1, cb1, cw2r)


def _node_mid_body(h_ref, agg0_ref, agg1_ref, s0_ref, s1_ref, cnt0_ref,
                   cnt1_ref, coord_ref,
                   nw1h_ref, nw1a_ref, nb1_ref, nw2_ref, nb2_ref,
                   sc_ref, bi_ref, a_ref, b_ref,
                   hn_ref, coordn_ref, ta_ref, tb_ref):
    h = h_ref[...]
    agg = agg0_ref[...] + agg1_ref[...]
    ssum = s0_ref[...] + s1_ref[...]
    cnt = cnt0_ref[...] + cnt1_ref[...]
    coordn_ref[...] = coord_ref[...] + ssum / jnp.maximum(cnt, 1.0)
    t = _silu(jnp.dot(h, nw1h_ref[...], preferred_element_type=f32)
              + jnp.dot(agg, nw1a_ref[...], preferred_element_type=f32)
              + nb1_ref[...])
    out = jnp.dot(t, nw2_ref[...], preferred_element_type=f32) + nb2_ref[...]
    hn = sc_ref[...] * (h + out) + bi_ref[...]
    hn_ref[...] = hn
    ta_ref[...] = jnp.dot(hn, a_ref[...], preferred_element_type=f32)
    tb_ref[...] = jnp.dot(hn, b_ref[...], preferred_element_type=f32)


def _node_mid(h, aggA, aggB, sA, sB, cntp, coord, nw1h, nw1a, nb1, nw2, nb2,
              scale, bias, a_w, b_w):
    return pl.pallas_call(
        _node_mid_body,
        grid=(NBLK,),
        in_specs=[
            _rowblk(H),
            _rowblk(H), _rowblk(H),
            _rowblk(H), _rowblk(H),
            _rowblk(H), _rowblk2(H),
            _rowblk(H),
            _full((H, H)), _full((H, H)), _full((1, H)),
            _full((H, H)), _full((1, H)),
            _full((1, H)), _full((1, H)),
            _full((H, H)), _full((H, H)),
        ],
        out_specs=[_rowblk(H), _rowblk(H), _rowblk(H), _rowblk(H)],
        out_shape=[
            jax.ShapeDtypeStruct((N, H), f32),
            jax.ShapeDtypeStruct((N, H), f32),
            jax.ShapeDtypeStruct((N, H), f32),
            jax.ShapeDtypeStruct((N, H), f32),
        ],
    )(h, aggA, aggB, sA, sB, cntp, cntp, coord, nw1h, nw1a, nb1, nw2, nb2,
      scale, bias, a_w, b_w)


def _node_last_body(h_ref, agg0_ref, agg1_ref, s0_ref, s1_ref, cnt0_ref,
                    cnt1_ref, coord_ref,
                    nw1h_ref, nw1a_ref, nb1_ref, nw2_ref, nb2_ref, batch_ref,
                    hn_ref, coordn_ref, gsum_ref, gcnt_ref):
    i = pl.program_id(0)
    h = h_ref[...]
    agg = agg0_ref[...] + agg1_ref[...]
    ssum = s0_ref[...] + s1_ref[...]
    cnt = cnt0_ref[...] + cnt1_ref[...]
    coordn_ref[...] = coord_ref[...] + ssum / jnp.maximum(cnt, 1.0)
    t = _silu(jnp.dot(h, nw1h_ref[...], preferred_element_type=f32)
              + jnp.dot(agg, nw1a_ref[...], preferred_element_type=f32)
              + nb1_ref[...])
    hend = (h + jnp.dot(t, nw2_ref[...], preferred_element_type=f32)
            + nb2_ref[...])
    hn_ref[...] = hend
    iota = lax.broadcasted_iota(i32, (NB, NGRAPH), 1)
    mask = (batch_ref[...] == iota).astype(f32)
    gsum_part = lax.dot_general(mask, hend, (((0,), (0,)), ((), ())),
                                preferred_element_type=f32)
    gcnt_part = jnp.sum(mask, axis=0, keepdims=True)

    @pl.when(i == 0)
    def _():
        gsum_ref[...] = jnp.zeros_like(gsum_ref)
        gcnt_ref[...] = jnp.zeros_like(gcnt_ref)

    gsum_ref[...] += gsum_part
    gcnt_ref[...] += gcnt_part


def _node_last(h, aggA, aggB, sA, sB, cntp, coord, nw1h, nw1a, nb1, nw2,
               nb2, batch2d):
    return pl.pallas_call(
        _node_last_body,
        grid=(NBLK,),
        in_specs=[
            _rowblk(H),
            _rowblk(H), _rowblk(H),
            _rowblk(H), _rowblk(H),
            _rowblk(H), _rowblk2(H),
            _rowblk(H),
            _full((H, H)), _full((H, H)), _full((1, H)),
            _full((H, H)), _full((1, H)),
            _rowblk(1),
        ],
        out_specs=[
            _rowblk(H),
            _rowblk(H),
            pl.BlockSpec((NGRAPH, H), lambda i: (jnp.int32(0), jnp.int32(0))),
            pl.BlockSpec((1, NGRAPH), lambda i: (jnp.int32(0), jnp.int32(0))),
        ],
        out_shape=[
            jax.ShapeDtypeStruct((N, H), f32),
            jax.ShapeDtypeStruct((N, H), f32),
            jax.ShapeDtypeStruct((NGRAPH, H), f32),
            jax.ShapeDtypeStruct((1, NGRAPH), f32),
        ],
    )(h, aggA, aggB, sA, sB, cntp, cntp, coord, nw1h, nw1a, nb1, nw2, nb2,
      batch2d)


def _pred_body(h_ref, gsum_ref, gcnt_ref, batch_ref, pw1g_ref, pw1h_ref,
               pb1_ref, pw2_ref, pb2_ref, pw3_ref, pb3_ref, out_ref):
    g = gsum_ref[...] / jnp.maximum(gcnt_ref[...].reshape(NGRAPH, 1), 1.0)
    iota = lax.broadcasted_iota(i32, (NB, NGRAPH), 1)
    mask = (batch_ref[...] == iota).astype(f32)
    gnode = jnp.dot(mask, g, preferred_element_type=f32)
    t = jnp.maximum(jnp.dot(gnode, pw1g_ref[...], preferred_element_type=f32)
                    + jnp.dot(h_ref[...], pw1h_ref[...],
                              preferred_element_type=f32)
                    + pb1_ref[...], 0.0)
    t = jnp.maximum(jnp.dot(t, pw2_ref[...], preferred_element_type=f32)
                    + pb2_ref[...], 0.0)
    out_ref[...] = (jnp.dot(t, pw3_ref[...], preferred_element_type=f32)
                    + pb3_ref[...])


def _pred_mlp(hend, gsum, gcnt, batch2d, pw1g, pw1h, pb1, pw2, pb2, pw3, pb3):
    return pl.pallas_call(
        _pred_body,
        grid=(NBLK,),
        in_specs=[
            _rowblk(H),
            _full((NGRAPH, H)),
            _full((1, NGRAPH)),
            _rowblk(1),
            _full((H, H)), _full((H, H)), _full((1, H)),
            _full((H, H)), _full((1, H)),
            _full((H, PRED * NF)), _full((1, PRED * NF)),
        ],
        out_specs=_rowblk(PRED * NF),
        out_shape=jax.ShapeDtypeStruct((N, PRED * NF), f32),
    )(hend, gsum, gcnt, batch2d, pw1g, pw1h, pb1, pw2, pb2, pw3, pb3)


def kernel(x, edge_attr, x_coord, film_cond, node_emb_w, node_emb_b,
           edge_emb_w, edge_emb_b, l_edge_w1, l_edge_b1, l_edge_w2, l_edge_b2,
           l_node_w1, l_node_b1, l_node_w2, l_node_b2, l_coord_w1, l_coord_b1,
           l_coord_w2, pred_w1, pred_b1, pred_w2, pred_b2, pred_w3, pred_b3,
           edge_index, batch):
    row = edge_index[0].astype(i32)
    col = edge_index[1].astype(i32)
    batch2d = batch.astype(i32).reshape(N, 1)
    xf = x.reshape(N, PRED * NF).astype(f32)
    ea2 = edge_attr.astype(f32).reshape(E, 1)
    coordp = jnp.zeros((N, H), f32).at[:, :3].set(x_coord.astype(f32))

    embed = film_cond.astype(f32).reshape(L, 2, H)
    scales = embed[:, 0, :]
    biases = embed[:, 1, :]

    zh = jnp.zeros((N, H), f32)
    ones_chunk = jnp.ones((CHUNK, H), f32)

    cntp = _sc_count(row, ones_chunk, zh)

    def r1(v):
        return v.astype(f32).reshape(1, -1)

    h, ta, tb = _node_init(
        xf, node_emb_w.astype(f32), r1(node_emb_b), scales[0:1], biases[0:1],
        l_edge_w1[0, :H].astype(f32), l_edge_w1[0, H:2 * H].astype(f32))
    coord = coordp

    rows_h = (row[:EH], row[EH:])
    cols_h = (col[:EH], col[EH:])
    eas_h = (ea2[:EH], ea2[EH:])

    for l in range(L):
        w1 = l_edge_w1[l].astype(f32)
        ew = (w1[2 * H:2 * H + 1, :], edge_emb_w.astype(f32), r1(edge_emb_b),
              w1[2 * H + 1:, :], r1(l_edge_b1[l]),
              l_edge_w2[l].astype(f32), r1(l_edge_b2[l]),
              l_coord_w1[l].astype(f32), r1(l_coord_b1[l]),
              l_coord_w2[l].astype(f32).reshape(1, H))
        ga0, gb0 = _sc_hgather(ta, tb, rows_h[0], cols_h[0])
        cd0 = _sc_cgather(coord, rows_h[0], cols_h[0])
        ga1, gb1 = _sc_hgather(ta, tb, rows_h[1], cols_h[1])
        cd1 = _sc_cgather(coord, rows_h[1], cols_h[1])
        feat0, trans0 = _edge_mlp(ga0, gb0, cd0, eas_h[0], *ew)
        feat1, trans1 = _edge_mlp(ga1, gb1, cd1, eas_h[1], *ew)
        aggA, sA = _sc_scatter2(feat0, trans0, rows_h[0], zh)
        aggB, sB = _sc_scatter2(feat1, trans1, rows_h[1], zh)
        nw1 = l_node_w1[l].astype(f32)
        if l < L - 1:
            h, coord, ta, tb = _node_mid(
                h, aggA, aggB, sA, sB, cntp, coord,
                nw1[:H], nw1[H:], r1(l_node_b1[l]),
                l_node_w2[l].astype(f32), r1(l_node_b2[l]),
                scales[l + 1:l + 2], biases[l + 1:l + 2],
                l_edge_w1[l + 1, :H].astype(f32),
                l_edge_w1[l + 1, H:2 * H].astype(f32))
        else:
            hend, coord, gsum, gcnt = _node_last(
                h, aggA, aggB, sA, sB, cntp, coord,
                nw1[:H], nw1[H:], r1(l_node_b1[l]),
                l_node_w2[l].astype(f32), r1(l_node_b2[l]), batch2d)

    p = _pred_mlp(hend, gsum, gcnt, batch2d,
                  pred_w1[:H].astype(f32), pred_w1[H:].astype(f32),
                  r1(pred_b1), pred_w2.astype(f32), r1(pred_b2),
                  pred_w3.astype(f32), r1(pred_b3))
    return p.reshape(N, PRED, NF), coord[:, :3]


# speedup vs baseline: 1.2023x; 1.0025x over previous
import functools

import jax
import jax.numpy as jnp
from jax import lax
from jax.experimental import pallas as pl
from jax.experimental.pallas import tpu as pltpu
from jax.experimental.pallas import tpu_sc as plsc

N = 10000
E = 160000
H = 128
L = 5
PRED = 16
NF = 4
NGRAPH = 16
EPS = 1e-8

NWORK = 32
CHUNK = 128
NCHUNK = E // CHUNK
ITERS = -(-NCHUNK // NWORK)
EH = E // 2
NCHUNK_H = EH // CHUNK
ITERS_H = 20
ITERS2_H = 40
NB = 1000
EB = 4000
NBLK = N // NB

f32 = jnp.float32
i32 = jnp.int32


def _silu(x):
    return x * (1.0 / (1.0 + jnp.exp(-x)))


def _pipeline(nwork, wid, niters, nchunk, start, finish):
    def guarded(j, fn, ph):
        k = j * jnp.int32(nwork) + wid

        @pl.when((j < niters) & (k < nchunk))
        def _():
            fn(k * jnp.int32(CHUNK), ph)

    guarded(jnp.int32(0), start, 0)

    def body(jj, carry):
        j0 = jj * jnp.int32(2)
        guarded(j0 + 1, start, 1)
        guarded(j0, finish, 0)
        guarded(j0 + 2, start, 0)
        guarded(j0 + 1, finish, 1)
        return carry

    lax.fori_loop(jnp.int32(0), jnp.int32(niters // 2), body, jnp.int32(0))


def _sc_hgather_body(nchunk, niters, ta_hbm, tb_hbm, row_hbm, col_hbm,
                     ga_hbm, gb_hbm,
                     idxr_v, idxc_v, bufa_v, bufb_v, sa0, sa1, sb0, sb1):
    wid = (lax.axis_index("s").astype(i32) * jnp.int32(2)
           + lax.axis_index("c").astype(i32))
    sems_a = (sa0, sa1)
    sems_b = (sb0, sb1)

    def start(base, ph):
        pltpu.sync_copy(row_hbm.at[pl.ds(base, CHUNK)], idxr_v.at[jnp.int32(ph)])
        pltpu.sync_copy(col_hbm.at[pl.ds(base, CHUNK)], idxc_v.at[jnp.int32(ph)])
        pltpu.async_copy(ta_hbm.at[idxr_v.at[jnp.int32(ph)]], bufa_v.at[jnp.int32(ph)], sems_a[ph])
        pltpu.async_copy(tb_hbm.at[idxc_v.at[jnp.int32(ph)]], bufb_v.at[jnp.int32(ph)], sems_b[ph])

    def finish(base, ph):
        pltpu.make_async_copy(ta_hbm.at[idxr_v.at[jnp.int32(ph)]], bufa_v.at[jnp.int32(ph)],
                              sems_a[ph]).wait()
        pltpu.make_async_copy(tb_hbm.at[idxc_v.at[jnp.int32(ph)]], bufb_v.at[jnp.int32(ph)],
                              sems_b[ph]).wait()
        pltpu.sync_copy(bufa_v.at[jnp.int32(ph)], ga_hbm.at[pl.ds(base, CHUNK)])
        pltpu.sync_copy(bufb_v.at[jnp.int32(ph)], gb_hbm.at[pl.ds(base, CHUNK)])

    _pipeline(NWORK, wid, niters, nchunk, start, finish)


def _sc_cgather_body(nchunk, niters, tc_hbm, row_hbm, col_hbm, cd_hbm,
                     idxr_v, idxc_v, bufr_v, bufc_v, sr0, sr1, sc0, sc1):
    wid = (lax.axis_index("s").astype(i32) * jnp.int32(2)
           + lax.axis_index("c").astype(i32))
    sems_r = (sr0, sr1)
    sems_c = (sc0, sc1)

    def start(base, ph):
        pltpu.sync_copy(row_hbm.at[pl.ds(base, CHUNK)], idxr_v.at[jnp.int32(ph)])
        pltpu.sync_copy(col_hbm.at[pl.ds(base, CHUNK)], idxc_v.at[jnp.int32(ph)])
        pltpu.async_copy(tc_hbm.at[idxr_v.at[jnp.int32(ph)]], bufr_v.at[jnp.int32(ph)], sems_r[ph])
        pltpu.async_copy(tc_hbm.at[idxc_v.at[jnp.int32(ph)]], bufc_v.at[jnp.int32(ph)], sems_c[ph])

    def finish(base, ph):
        pltpu.make_async_copy(tc_hbm.at[idxr_v.at[jnp.int32(ph)]], bufr_v.at[jnp.int32(ph)],
                              sems_r[ph]).wait()
        pltpu.make_async_copy(tc_hbm.at[idxc_v.at[jnp.int32(ph)]], bufc_v.at[jnp.int32(ph)],
                              sems_c[ph]).wait()

        def sub_row(r, carry):
            a = bufr_v[jnp.int32(ph), r, pl.ds(0, 16)]
            b = bufc_v[jnp.int32(ph), r, pl.ds(0, 16)]
            bufr_v[jnp.int32(ph), r, pl.ds(0, 16)] = a - b
            return carry

        lax.fori_loop(jnp.int32(0), jnp.int32(CHUNK), sub_row, jnp.int32(0))
        pltpu.sync_copy(bufr_v.at[jnp.int32(ph)], cd_hbm.at[pl.ds(base, CHUNK)])

    _pipeline(NWORK, wid, niters, nchunk, start, finish)


def _writeout(sid, src_sh, dst_hbm):
    @pl.when(sid < 15)
    def _():
        off = sid * jnp.int32(624)
        pltpu.sync_copy(src_sh.at[pl.ds(off, 624)],
                        dst_hbm.at[pl.ds(off, 624)])

    @pl.when(sid == 15)
    def _():
        pltpu.sync_copy(src_sh.at[pl.ds(9360, 640)],
                        dst_hbm.at[pl.ds(jnp.int32(9360), 640)])


def _sc_scatter2_body(nchunk, niters, feat_hbm, trans_hbm, row_hbm, zh_hbm,
                      agg_hbm, s_hbm, idx_v, dat_v, acc_sh, s0, s1):
    cid = lax.axis_index("c").astype(i32)
    sid = lax.axis_index("s").astype(i32)
    sems = (s0, s1)

    @pl.when(sid == 0)
    def _():
        pltpu.sync_copy(zh_hbm, acc_sh)

    plsc.subcore_barrier()

    def run(src_hbm):
        def start(base, ph):
            pltpu.sync_copy(row_hbm.at[pl.ds(base, CHUNK)], idx_v.at[jnp.int32(ph)])
            pltpu.async_copy(src_hbm.at[pl.ds(base, CHUNK)], dat_v.at[jnp.int32(ph)],
                             sems[ph])

        def finish(base, ph):
            pltpu.make_async_copy(src_hbm.at[pl.ds(base, CHUNK)],
                                  dat_v.at[jnp.int32(ph)], sems[ph]).wait()
            pltpu.sync_copy(dat_v.at[jnp.int32(ph)], acc_sh.at[idx_v.at[jnp.int32(ph)]], add=True)

        _pipeline(16, sid, niters, nchunk, start, finish)

    @pl.when(cid == 0)
    def _():
        run(feat_hbm)

    @pl.when(cid == 1)
    def _():
        run(trans_hbm)

    plsc.subcore_barrier()

    @pl.when(cid == 0)
    def _():
        _writeout(sid, acc_sh, agg_hbm)

    @pl.when(cid == 1)
    def _():
        _writeout(sid, acc_sh, s_hbm)


def _sc_count_body(row_hbm, ones_hbm, zh_hbm, cnt_hbm, idx_v, ones_v, cnt_sh):
    cid = lax.axis_index("c").astype(i32)
    sid = lax.axis_index("s").astype(i32)
    wid = sid * jnp.int32(2) + cid

    @pl.when(sid == 0)
    def _():
        pltpu.sync_copy(zh_hbm, cnt_sh)

    pltpu.sync_copy(ones_hbm, ones_v)
    plsc.subcore_barrier()

    def body(j, carry):
        k = j * jnp.int32(NWORK) + wid

        @pl.when(k < NCHUNK)
        def _():
            base = k * jnp.int32(CHUNK)
            pltpu.sync_copy(row_hbm.at[pl.ds(base, CHUNK)], idx_v)
            pltpu.sync_copy(ones_v, cnt_sh.at[idx_v], add=True)

        return carry

    lax.fori_loop(jnp.int32(0), jnp.int32(ITERS), body, jnp.int32(0))
    plsc.subcore_barrier()

    @pl.when(cid == 0)
    def _():
        _writeout(sid, cnt_sh, cnt_hbm.at[pl.ds(0, N)])

    @pl.when(cid == 1)
    def _():
        _writeout(sid, cnt_sh, cnt_hbm.at[pl.ds(N, N)])


@functools.cache
def _sc_kernels():
    mesh = plsc.VectorSubcoreMesh(core_axis_name="c", subcore_axis_name="s",
                                  num_cores=2, num_subcores=16)
    hgather = pl.kernel(
        functools.partial(_sc_hgather_body, NCHUNK_H, ITERS_H),
        out_type=(
            jax.ShapeDtypeStruct((EH, H), f32),
            jax.ShapeDtypeStruct((EH, H), f32),
        ),
        mesh=mesh,
        scratch_types=[
            pltpu.VMEM((2, CHUNK), i32),
            pltpu.VMEM((2, CHUNK), i32),
            pltpu.VMEM((2, CHUNK, H), f32),
            pltpu.VMEM((2, CHUNK, H), f32),
            pltpu.SemaphoreType.DMA,
            pltpu.SemaphoreType.DMA,
            pltpu.SemaphoreType.DMA,
            pltpu.SemaphoreType.DMA,
        ],
    )
    cgather = pl.kernel(
        functools.partial(_sc_cgather_body, NCHUNK_H, ITERS_H),
        out_type=jax.ShapeDtypeStruct((EH, H), f32),
        mesh=mesh,
        scratch_types=[
            pltpu.VMEM((2, CHUNK), i32),
            pltpu.VMEM((2, CHUNK), i32),
            pltpu.VMEM((2, CHUNK, H), f32),
            pltpu.VMEM((2, CHUNK, H), f32),
            pltpu.SemaphoreType.DMA,
            pltpu.SemaphoreType.DMA,
            pltpu.SemaphoreType.DMA,
            pltpu.SemaphoreType.DMA,
        ],
    )
    scatter2 = pl.kernel(
        functools.partial(_sc_scatter2_body, NCHUNK_H, ITERS2_H),
        out_type=(
            jax.ShapeDtypeStruct((N, H), f32),
            jax.ShapeDtypeStruct((N, H), f32),
        ),
        mesh=mesh,
        scratch_types=[
            pltpu.VMEM((2, CHUNK), i32),
            pltpu.VMEM((2, CHUNK, H), f32),
            pltpu.VMEM_SHARED((N, H), f32),
            pltpu.SemaphoreType.DMA,
            pltpu.SemaphoreType.DMA,
        ],
    )
    count = pl.kernel(
        _sc_count_body,
        out_type=jax.ShapeDtypeStruct((2 * N, H), f32),
        mesh=mesh,
        scratch_types=[
            pltpu.VMEM((CHUNK,), i32),
            pltpu.VMEM((CHUNK, H), f32),
            pltpu.VMEM_SHARED((N, H), f32),
        ],
    )
    return hgather, cgather, scatter2, count


def _sc_hgather(ta, tb, row, col):
    return _sc_kernels()[0](ta, tb, row, col)


def _sc_cgather(coordp, row, col):
    return _sc_kernels()[1](coordp, row, col)


def _sc_scatter2(feat, trans, row, zh):
    return _sc_kernels()[2](feat, trans, row, zh)


def _sc_count(row, ones_chunk, zh):
    return _sc_kernels()[3](row, ones_chunk, zh)


def _full(shape):
    return pl.BlockSpec(shape, lambda i: tuple(jnp.int32(0) for _ in shape))


def _rowblk(cols):
    return pl.BlockSpec((NB, cols), lambda i: (i, jnp.int32(0)))


def _rowblk2(cols):
    return pl.BlockSpec((NB, cols),
                        lambda i: (i + jnp.int32(NBLK), jnp.int32(0)))


def _node_init_body(xf_ref, wemb_ref, bemb_ref, sc_ref, bi_ref,
                    a_ref, b_ref, h_ref, ta_ref, tb_ref):
    h0 = jnp.dot(xf_ref[...], wemb_ref[...], preferred_element_type=f32)
    h = sc_ref[...] * (h0 + bemb_ref[...]) + bi_ref[...]
    h_ref[...] = h
    ta_ref[...] = jnp.dot(h, a_ref[...], preferred_element_type=f32)
    tb_ref[...] = jnp.dot(h, b_ref[...], preferred_element_type=f32)


def _node_init(xf, wemb, bemb, scale, bias, a_w, b_w):
    return pl.pallas_call(
        _node_init_body,
        grid=(NBLK,),
        in_specs=[
            _rowblk(PRED * NF),
            _full((PRED * NF, H)),
            _full((1, H)),
            _full((1, H)),
            _full((1, H)),
            _full((H, H)),
            _full((H, H)),
        ],
        out_specs=[_rowblk(H), _rowblk(H), _rowblk(H)],
        out_shape=[
            jax.ShapeDtypeStruct((N, H), f32),
            jax.ShapeDtypeStruct((N, H), f32),
            jax.ShapeDtypeStruct((N, H), f32),
        ],
    )(xf, wemb, bemb, scale, bias, a_w, b_w)


def _edge_body(ga_ref, gb_ref, cd_ref, ea_ref, uvec_ref, eew_ref,
               eeb_ref, w1he_ref, b1_ref, w2_ref, b2_ref, cw1_ref, cb1_ref,
               cw2_ref, feat_ref, trans_ref):
    hpart = ga_ref[...] + gb_ref[...]
    cd = cd_ref[...]
    radial = jnp.sum(cd * cd, axis=1, keepdims=True)
    norm = jnp.sqrt(radial) + EPS
    cdn = cd / norm
    w1he = w1he_ref[...]
    vvec = jnp.dot(eew_ref[...], w1he, preferred_element_type=f32)
    cvec = jnp.dot(eeb_ref[...], w1he, preferred_element_type=f32) + b1_ref[...]
    pre1 = hpart + radial * uvec_ref[...] + ea_ref[...] * vvec + cvec
    f1 = _silu(pre1)
    f2 = _silu(jnp.dot(f1, w2_ref[...], preferred_element_type=f32)
               + b2_ref[...])
    hc = _silu(jnp.dot(f2, cw1_ref[...], preferred_element_type=f32)
               + cb1_ref[...])
    cm = jnp.sum(hc * cw2_ref[...], axis=1, keepdims=True)
    feat_ref[...] = f2
    trans_ref[...] = cdn * cm


def _edge_mlp(ga, gb, cd, ea2, uvec, eew, eeb, w1he, b1, w2, b2,
              cw1, cb1, cw2r):
    ne = ga.shape[0]
    eb = pl.BlockSpec((EB, H), lambda i: (i, jnp.int32(0)))
    eb1 = pl.BlockSpec((EB, 1), lambda i: (i, jnp.int32(0)))
    return pl.pallas_call(
        _edge_body,
        grid=(ne // EB,),
        in_specs=[
            eb, eb, eb, eb1,
            _full((1, H)), _full((1, H)), _full((1, H)),
            _full((H, H)), _full((1, H)),
            _full((H, H)), _full((1, H)),
            _full((H, H)), _full((1, H)), _full((1, H)),
        ],
        out_specs=[eb, eb],
        out_shape=[
            jax.ShapeDtypeStruct((ne, H), f32),
            jax.ShapeDtypeStruct((ne, H), f32),
        ],
    )(ga, gb, cd, ea2, uvec, eew, eeb, w1he, b1, w2, b2, cw1, cb1, cw2r)


def _node_mid_body(h_ref, agg0_ref, agg1_ref, s0_ref, s1_ref, cnt0_ref,
                   cnt1_ref, coord_ref,
                   nw1h_ref, nw1a_ref, nb1_ref, nw2_ref, nb2_ref,
                   sc_ref, bi_ref, a_ref, b_ref,
                   hn_ref, coordn_ref, ta_ref, tb_ref):
    h = h_ref[...]
    agg = agg0_ref[...] + agg1_ref[...]
    ssum = s0_ref[...] + s1_ref[...]
    cnt = cnt0_ref[...] + cnt1_ref[...]
    coordn_ref[...] = coord_ref[...] + ssum / jnp.maximum(cnt, 1.0)
    t = _silu(jnp.dot(h, nw1h_ref[...], preferred_element_type=f32)
              + jnp.dot(agg, nw1a_ref[...], preferred_element_type=f32)
              + nb1_ref[...])
    out = jnp.dot(t, nw2_ref[...], preferred_element_type=f32) + nb2_ref[...]
    hn = sc_ref[...] * (h + out) + bi_ref[...]
    hn_ref[...] = hn
    ta_ref[...] = jnp.dot(hn, a_ref[...], preferred_element_type=f32)
    tb_ref[...] = jnp.dot(hn, b_ref[...], preferred_element_type=f32)


def _node_mid(h, aggA, aggB, sA, sB, cntp, coord, nw1h, nw1a, nb1, nw2, nb2,
              scale, bias, a_w, b_w):
    return pl.pallas_call(
        _node_mid_body,
        grid=(NBLK,),
        in_specs=[
            _rowblk(H),
            _rowblk(H), _rowblk(H),
            _rowblk(H), _rowblk(H),
            _rowblk(H), _rowblk2(H),
            _rowblk(H),
            _full((H, H)), _full((H, H)), _full((1, H)),
            _full((H, H)), _full((1, H)),
            _full((1, H)), _full((1, H)),
            _full((H, H)), _full((H, H)),
        ],
        out_specs=[_rowblk(H), _rowblk(H), _rowblk(H), _rowblk(H)],
        out_shape=[
            jax.ShapeDtypeStruct((N, H), f32),
            jax.ShapeDtypeStruct((N, H), f32),
            jax.ShapeDtypeStruct((N, H), f32),
            jax.ShapeDtypeStruct((N, H), f32),
        ],
    )(h, aggA, aggB, sA, sB, cntp, cntp, coord, nw1h, nw1a, nb1, nw2, nb2,
      scale, bias, a_w, b_w)


def _node_last_body(h_ref, agg0_ref, agg1_ref, s0_ref, s1_ref, cnt0_ref,
                    cnt1_ref, coord_ref,
                    nw1h_ref, nw1a_ref, nb1_ref, nw2_ref, nb2_ref, batch_ref,
                    hn_ref, coordn_ref, gsum_ref, gcnt_ref):
    i = pl.program_id(0)
    h = h_ref[...]
    agg = agg0_ref[...] + agg1_ref[...]
    ssum = s0_ref[...] + s1_ref[...]
    cnt = cnt0_ref[...] + cnt1_ref[...]
    coordn_ref[...] = coord_ref[...] + ssum / jnp.maximum(cnt, 1.0)
    t = _silu(jnp.dot(h, nw1h_ref[...], preferred_element_type=f32)
              + jnp.dot(agg, nw1a_ref[...], preferred_element_type=f32)
              + nb1_ref[...])
    hend = (h + jnp.dot(t, nw2_ref[...], preferred_element_type=f32)
            + nb2_ref[...])
    hn_ref[...] = hend
    iota = lax.broadcasted_iota(i32, (NB, NGRAPH), 1)
    mask = (batch_ref[...] == iota).astype(f32)
    gsum_part = lax.dot_general(mask, hend, (((0,), (0,)), ((), ())),
                                preferred_element_type=f32)
    gcnt_part = jnp.sum(mask, axis=0, keepdims=True)

    @pl.when(i == 0)
    def _():
        gsum_ref[...] = jnp.zeros_like(gsum_ref)
        gcnt_ref[...] = jnp.zeros_like(gcnt_ref)

    gsum_ref[...] += gsum_part
    gcnt_ref[...] += gcnt_part


def _node_last(h, aggA, aggB, sA, sB, cntp, coord, nw1h, nw1a, nb1, nw2,
               nb2, batch2d):
    return pl.pallas_call(
        _node_last_body,
        grid=(NBLK,),
        in_specs=[
            _rowblk(H),
            _rowblk(H), _rowblk(H),
            _rowblk(H), _rowblk(H),
            _rowblk(H), _rowblk2(H),
            _rowblk(H),
            _full((H, H)), _full((H, H)), _full((1, H)),
            _full((H, H)), _full((1, H)),
            _rowblk(1),
        ],
        out_specs=[
            _rowblk(H),
            _rowblk(H),
            pl.BlockSpec((NGRAPH, H), lambda i: (jnp.int32(0), jnp.int32(0))),
            pl.BlockSpec((1, NGRAPH), lambda i: (jnp.int32(0), jnp.int32(0))),
        ],
        out_shape=[
            jax.ShapeDtypeStruct((N, H), f32),
            jax.ShapeDtypeStruct((N, H), f32),
            jax.ShapeDtypeStruct((NGRAPH, H), f32),
            jax.ShapeDtypeStruct((1, NGRAPH), f32),
        ],
    )(h, aggA, aggB, sA, sB, cntp, cntp, coord, nw1h, nw1a, nb1, nw2, nb2,
      batch2d)


def _pred_body(h_ref, gsum_ref, gcnt_ref, batch_ref, pw1g_ref, pw1h_ref,
               pb1_ref, pw2_ref, pb2_ref, pw3_ref, pb3_ref, out_ref):
    g = gsum_ref[...] / jnp.maximum(gcnt_ref[...].reshape(NGRAPH, 1), 1.0)
    iota = lax.broadcasted_iota(i32, (NB, NGRAPH), 1)
    mask = (batch_ref[...] == iota).astype(f32)
    gnode = jnp.dot(mask, g, preferred_element_type=f32)
    t = jnp.maximum(jnp.dot(gnode, pw1g_ref[...], preferred_element_type=f32)
                    + jnp.dot(h_ref[...], pw1h_ref[...],
                              preferred_element_type=f32)
                    + pb1_ref[...], 0.0)
    t = jnp.maximum(jnp.dot(t, pw2_ref[...], preferred_element_type=f32)
                    + pb2_ref[...], 0.0)
    out_ref[...] = (jnp.dot(t, pw3_ref[...], preferred_element_type=f32)
                    + pb3_ref[...])


def _pred_mlp(hend, gsum, gcnt, batch2d, pw1g, pw1h, pb1, pw2, pb2, pw3, pb3):
    return pl.pallas_call(
        _pred_body,
        grid=(NBLK,),
        in_specs=[
            _rowblk(H),
            _full((NGRAPH, H)),
            _full((1, NGRAPH)),
            _rowblk(1),
            _full((H, H)), _full((H, H)), _full((1, H)),
            _full((H, H)), _full((1, H)),
            _full((H, PRED * NF)), _full((1, PRED * NF)),
        ],
        out_specs=_rowblk(PRED * NF),
        out_shape=jax.ShapeDtypeStruct((N, PRED * NF), f32),
    )(hend, gsum, gcnt, batch2d, pw1g, pw1h, pb1, pw2, pb2, pw3, pb3)


def kernel(x, edge_attr, x_coord, film_cond, node_emb_w, node_emb_b,
           edge_emb_w, edge_emb_b, l_edge_w1, l_edge_b1, l_edge_w2, l_edge_b2,
           l_node_w1, l_node_b1, l_node_w2, l_node_b2, l_coord_w1, l_coord_b1,
           l_coord_w2, pred_w1, pred_b1, pred_w2, pred_b2, pred_w3, pred_b3,
           edge_index, batch):
    row = edge_index[0].astype(i32)
    col = edge_index[1].astype(i32)
    batch2d = batch.astype(i32).reshape(N, 1)
    xf = x.reshape(N, PRED * NF).astype(f32)
    ea2 = edge_attr.astype(f32).reshape(E, 1)
    coordp = jnp.zeros((N, H), f32).at[:, :3].set(x_coord.astype(f32))

    embed = film_cond.astype(f32).reshape(L, 2, H)
    scales = embed[:, 0, :]
    biases = embed[:, 1, :]

    zh = jnp.zeros((N, H), f32)
    ones_chunk = jnp.ones((CHUNK, H), f32)

    cntp = _sc_count(row, ones_chunk, zh)

    def r1(v):
        return v.astype(f32).reshape(1, -1)

    h, ta, tb = _node_init(
        xf, node_emb_w.astype(f32), r1(node_emb_b), scales[0:1], biases[0:1],
        l_edge_w1[0, :H].astype(f32), l_edge_w1[0, H:2 * H].astype(f32))
    coord = coordp

    rows_h = (row[:EH], row[EH:])
    cols_h = (col[:EH], col[EH:])
    eas_h = (ea2[:EH], ea2[EH:])

    for l in range(L):
        w1 = l_edge_w1[l].astype(f32)
        ew = (w1[2 * H:2 * H + 1, :], edge_emb_w.astype(f32), r1(edge_emb_b),
              w1[2 * H + 1:, :], r1(l_edge_b1[l]),
              l_edge_w2[l].astype(f32), r1(l_edge_b2[l]),
              l_coord_w1[l].astype(f32), r1(l_coord_b1[l]),
              l_coord_w2[l].astype(f32).reshape(1, H))
        ga0, gb0 = _sc_hgather(ta, tb, rows_h[0], cols_h[0])
        cd0 = _sc_cgather(coord, rows_h[0], cols_h[0])
        ga1, gb1 = _sc_hgather(ta, tb, rows_h[1], cols_h[1])
        cd1 = _sc_cgather(coord, rows_h[1], cols_h[1])
        feat0, trans0 = _edge_mlp(ga0, gb0, cd0, eas_h[0], *ew)
        feat1, trans1 = _edge_mlp(ga1, gb1, cd1, eas_h[1], *ew)
        aggA, sA = _sc_scatter2(feat0, trans0, rows_h[0], zh)
        aggB, sB = _sc_scatter2(feat1, trans1, rows_h[1], zh)
        nw1 = l_node_w1[l].astype(f32)
        if l < L - 1:
            h, coord, ta, tb = _node_mid(
                h, aggA, aggB, sA, sB, cntp, coord,
                nw1[:H], nw1[H:], r1(l_node_b1[l]),
                l_node_w2[l].astype(f32), r1(l_node_b2[l]),
                scales[l + 1:l + 2], biases[l + 1:l + 2],
                l_edge_w1[l + 1, :H].astype(f32),
                l_edge_w1[l + 1, H:2 * H].astype(f32))
        else:
            hend, coord, gsum, gcnt = _node_last(
                h, aggA, aggB, sA, sB, cntp, coord,
                nw1[:H], nw1[H:], r1(l_node_b1[l]),
                l_node_w2[l].astype(f32), r1(l_node_b2[l]), batch2d)

    p = _pred_mlp(hend, gsum, gcnt, batch2d,
                  pred_w1[:H].astype(f32), pred_w1[H:].astype(f32),
                  r1(pred_b1), pred_w2.astype(f32), r1(pred_b2),
                  pred_w3.astype(f32), r1(pred_b3))
    return p.reshape(N, PRED, NF), coord[:, :3]
